# Initial kernel scaffold; baseline (speedup 1.0000x reference)
import functools

import jax
import jax.numpy as jnp
from jax import lax
from jax.experimental import pallas as pl
from jax.experimental.pallas import tpu as pltpu
from jax.experimental.pallas import tpu_sc as plsc

N = 10000
E = 320000
D = 128
C = 40
CP = 64

NC = 2
NS = 16
NW = NC * NS
EPW = E // NW
CHUNK = 80
NCHUNK = EPW // CHUNK
NPS = N // NS

_MESH = plsc.VectorSubcoreMesh(core_axis_name="c", subcore_axis_name="s")


def _deg_body(src_hbm, dst_hbm, zeros_hbm, out_hbm,
              src_v, dst_v, ones_v, dout_sh, din_sh):
    cid = lax.axis_index("c")
    sid = lax.axis_index("s")
    wid = sid * NC + cid

    @pl.when(sid == 0)
    def _():
        pltpu.sync_copy(zeros_hbm.at[pl.ds(0, N)], dout_sh)

    @pl.when(sid == 1)
    def _():
        pltpu.sync_copy(zeros_hbm.at[pl.ds(0, N)], din_sh)

    pltpu.sync_copy(src_hbm.at[wid], src_v)
    pltpu.sync_copy(dst_hbm.at[wid], dst_v)
    for k in range(CHUNK // 16):
        ones_v[pl.ds(k * 16, 16)] = jnp.full((16,), 1.0, jnp.float32)
    plsc.subcore_barrier()

    def body(j, carry):
        pltpu.sync_copy(ones_v, dout_sh.at[src_v.at[j]], add=True)
        pltpu.sync_copy(ones_v, din_sh.at[dst_v.at[j]], add=True)
        return carry

    lax.fori_loop(0, NCHUNK, body, 0)
    plsc.subcore_barrier()

    @pl.when(sid == 0)
    def _():
        pltpu.sync_copy(dout_sh, out_hbm.at[cid, 0])

    @pl.when(sid == 1)
    def _():
        pltpu.sync_copy(din_sh, out_hbm.at[cid, 1])


_deg_kernel = functools.partial(
    pl.kernel,
    out_type=jax.ShapeDtypeStruct((NC, 2, N), jnp.float32),
    mesh=_MESH,
    scratch_types=[
        pltpu.VMEM((NCHUNK, CHUNK), jnp.int32),
        pltpu.VMEM((NCHUNK, CHUNK), jnp.int32),
        pltpu.VMEM((CHUNK,), jnp.float32),
        pltpu.VMEM_SHARED((N,), jnp.float32),
        pltpu.VMEM_SHARED((N,), jnp.float32),
    ],
)(_deg_body)


def _edge_body(dd, table_hbm, src_hbm, dst_hbm, zeros_hbm, out_hbm,
               src_v, dst_v, buf_v, agg_sh, sem):
    cid = lax.axis_index("c")
    sid = lax.axis_index("s")
    wid = sid * NC + cid

    pltpu.sync_copy(zeros_hbm.at[pl.ds(sid * NPS, NPS)],
                    agg_sh.at[pl.ds(sid * NPS, NPS)])
    pltpu.sync_copy(src_hbm.at[wid], src_v)
    pltpu.sync_copy(dst_hbm.at[wid], dst_v)
    plsc.subcore_barrier()

    def body(j, carry):
        pltpu.async_copy(table_hbm.at[src_v.at[j]], buf_v, sem).wait()
        pltpu.sync_copy(buf_v, agg_sh.at[dst_v.at[j]], add=True)
        return carry

    lax.fori_loop(0, NCHUNK, body, 0)
    plsc.subcore_barrier()
    pltpu.sync_copy(agg_sh.at[pl.ds(sid * NPS, NPS)],
                    out_hbm.at[cid, pl.ds(sid * NPS, NPS)])


def _make_edge_kernel(dd):
    return functools.partial(
        pl.kernel,
        out_type=jax.ShapeDtypeStruct((NC, N, dd), jnp.float32),
        mesh=_MESH,
        scratch_types=[
            pltpu.VMEM((NCHUNK, CHUNK), jnp.int32),
            pltpu.VMEM((NCHUNK, CHUNK), jnp.int32),
            pltpu.VMEM((CHUNK, dd), jnp.float32),
            pltpu.VMEM_SHARED((N, dd), jnp.float32),
            pltpu.SemaphoreType.DMA,
        ],
    )(functools.partial(_edge_body, dd))


_edge128 = _make_edge_kernel(D)
_edge64 = _make_edge_kernel(CP)


def _norms_body(degp_ref, ns_ref, nd_ref):
    dout = degp_ref[0, 0] + degp_ref[1, 0]
    din = degp_ref[0, 1] + degp_ref[1, 1]
    ns_ref[...] = jnp.where(dout > 0, lax.rsqrt(dout), 0.0)
    nd_ref[...] = jnp.where(din > 0, lax.rsqrt(din), 0.0)


def _norms(degp):
    return pl.pallas_call(
        _norms_body,
        out_shape=(jax.ShapeDtypeStruct((N, 1), jnp.float32),
                   jax.ShapeDtypeStruct((N, 1), jnp.float32)),
    )(degp.reshape(NC, 2, N, 1))


def _first_body(feats_ref, ns_ref, w_ref, out_ref):
    h = feats_ref[...] * ns_ref[...]
    out_ref[...] = jnp.dot(h, w_ref[...], preferred_element_type=jnp.float32)


def _first_tc(feats, ns, w):
    return pl.pallas_call(
        _first_body,
        out_shape=jax.ShapeDtypeStruct((N, D), jnp.float32),
    )(feats, ns, w)


def _mid_body(aggp_ref, nd_ref, ns_ref, w_ref, b_ref, sb_ref, out_ref):
    agg = aggp_ref[0] + aggp_ref[1]
    h = agg * nd_ref[...] + b_ref[...] + sb_ref[0, 0]
    h = jnp.maximum(h, 0.0) * ns_ref[...]
    out_ref[...] = jnp.dot(h, w_ref[...], preferred_element_type=jnp.float32)


def _mid_tc(aggp, nd, ns, w, b, sb):
    dout = w.shape[1]
    return pl.pallas_call(
        _mid_body,
        out_shape=jax.ShapeDtypeStruct((N, dout), jnp.float32),
    )(aggp, nd, ns, w, b.reshape(1, -1), sb.reshape(1, 1))


def _final_body(aggp_ref, nd_ref, b_ref, sb_ref, out_ref):
    agg = aggp_ref[0] + aggp_ref[1]
    h = agg * nd_ref[...] + b_ref[...] + sb_ref[0, 0]
    h = jnp.maximum(h, 0.0)
    col = lax.broadcasted_iota(jnp.int32, h.shape, 1)
    h = jnp.where(col < C, h, -1e30)
    m = jnp.max(h, axis=1, keepdims=True)
    e = jnp.exp(h - m)
    p = e / jnp.sum(e, axis=1, keepdims=True)
    out_ref[...] = p[:, :C]


def _final_tc(aggp, nd, b, sb):
    return pl.pallas_call(
        _final_body,
        out_shape=jax.ShapeDtypeStruct((N, C), jnp.float32),
    )(aggp, nd, b.reshape(1, -1), sb.reshape(1, 1))


def kernel(feats, edge_index, Ws, W_last, bs, b_last, sbs):
    ei = edge_index.astype(jnp.int32)
    src3 = ei[0].reshape(NW, NCHUNK, CHUNK)
    dst3 = ei[1].reshape(NW, NCHUNK, CHUNK)
    zeros_nd = jnp.zeros((N, D), jnp.float32)
    zeros_np = jnp.zeros((N, CP), jnp.float32)
    w_last_p = jnp.pad(W_last, ((0, 0), (0, CP - C)))
    b_last_p = jnp.pad(b_last, (0, CP - C))

    degp = _deg_kernel(src3, dst3, zeros_nd[:, 0])
    ns, nd = _norms(degp)

    table = _first_tc(feats, ns, Ws[0])
    aggp = _edge128(table, src3, dst3, zeros_nd)
    for i in range(1, 9):
        table = _mid_tc(aggp, nd, ns, Ws[i], bs[i - 1], sbs[i - 1])
        aggp = _edge128(table, src3, dst3, zeros_nd)
    table = _mid_tc(aggp, nd, ns, w_last_p, bs[8], sbs[8])
    aggp = _edge64(table, src3, dst3, zeros_np)
    return _final_tc(aggp, nd, b_last_p, sbs[9])


# speedup vs baseline: 6.4228x; 6.4228x over previous
import functools

import jax
import jax.numpy as jnp
from jax import lax
from jax.experimental import pallas as pl
from jax.experimental.pallas import tpu as pltpu
from jax.experimental.pallas import tpu_sc as plsc

N = 10000
E = 320000
D = 128
C = 40
CP = 128

NC = 2
NS = 16
NW = NC * NS
EPW = E // NW
CHUNK = 80
NCHUNK = EPW // CHUNK
NPS_A = 624
NPS_LAST = N - 15 * NPS_A

_MESH = plsc.VectorSubcoreMesh(core_axis_name="c", subcore_axis_name="s")


def _deg_body(src_hbm, dst_hbm, zeros_hbm, out_hbm,
              src_v, dst_v, ones_v, dout_sh, din_sh):
    cid = lax.axis_index("c")
    sid = lax.axis_index("s")
    wid = sid * NC + cid

    @pl.when(sid == 0)
    def _():
        pltpu.sync_copy(zeros_hbm.at[pl.ds(0, N)], dout_sh)

    @pl.when(sid == 1)
    def _():
        pltpu.sync_copy(zeros_hbm.at[pl.ds(0, N)], din_sh)

    pltpu.sync_copy(src_hbm.at[wid], src_v)
    pltpu.sync_copy(dst_hbm.at[wid], dst_v)
    for k in range(CHUNK // 16):
        ones_v[pl.ds(k * 16, 16)] = jnp.full((16,), 1.0, jnp.float32)
    plsc.subcore_barrier()

    def body(j, carry):
        pltpu.sync_copy(ones_v, dout_sh.at[src_v.at[j]], add=True)
        pltpu.sync_copy(ones_v, din_sh.at[dst_v.at[j]], add=True)
        return carry

    lax.fori_loop(0, NCHUNK, body, 0)
    plsc.subcore_barrier()

    @pl.when(sid == 0)
    def _():
        pltpu.sync_copy(dout_sh, out_hbm.at[cid, 0])

    @pl.when(sid == 1)
    def _():
        pltpu.sync_copy(din_sh, out_hbm.at[cid, 1])


_deg_kernel = functools.partial(
    pl.kernel,
    out_type=jax.ShapeDtypeStruct((NC, 2, N), jnp.float32),
    mesh=_MESH,
    scratch_types=[
        pltpu.VMEM((NCHUNK, CHUNK), jnp.int32),
        pltpu.VMEM((NCHUNK, CHUNK), jnp.int32),
        pltpu.VMEM((CHUNK,), jnp.float32),
        pltpu.VMEM_SHARED((N,), jnp.float32),
        pltpu.VMEM_SHARED((N,), jnp.float32),
    ],
)(_deg_body)


def _edge_body(dd, table_hbm, src_hbm, dst_hbm, zeros_hbm, out_hbm,
               src_v, dst_v, buf_v, agg_sh, sem):
    cid = lax.axis_index("c")
    sid = lax.axis_index("s")
    wid = sid * NC + cid

    base = pl.multiple_of(sid * NPS_A, 8)

    @pl.when(sid < 15)
    def _():
        pltpu.sync_copy(zeros_hbm.at[pl.ds(base, NPS_A)],
                        agg_sh.at[pl.ds(base, NPS_A)])

    @pl.when(sid == 15)
    def _():
        pltpu.sync_copy(zeros_hbm.at[pl.ds(15 * NPS_A, NPS_LAST)],
                        agg_sh.at[pl.ds(15 * NPS_A, NPS_LAST)])

    pltpu.sync_copy(src_hbm.at[wid], src_v)
    pltpu.sync_copy(dst_hbm.at[wid], dst_v)
    plsc.subcore_barrier()

    def body(j, carry):
        pltpu.async_copy(table_hbm.at[src_v.at[j]], buf_v, sem).wait()
        pltpu.sync_copy(buf_v, agg_sh.at[dst_v.at[j]], add=True)
        return carry

    lax.fori_loop(0, NCHUNK, body, 0)
    plsc.subcore_barrier()

    @pl.when(sid < 15)
    def _():
        pltpu.sync_copy(agg_sh.at[pl.ds(base, NPS_A)],
                        out_hbm.at[cid, pl.ds(base, NPS_A)])

    @pl.when(sid == 15)
    def _():
        pltpu.sync_copy(agg_sh.at[pl.ds(15 * NPS_A, NPS_LAST)],
                        out_hbm.at[cid, pl.ds(15 * NPS_A, NPS_LAST)])


def _make_edge_kernel(dd):
    return functools.partial(
        pl.kernel,
        out_type=jax.ShapeDtypeStruct((NC, N, dd), jnp.float32),
        mesh=_MESH,
        scratch_types=[
            pltpu.VMEM((NCHUNK, CHUNK), jnp.int32),
            pltpu.VMEM((NCHUNK, CHUNK), jnp.int32),
            pltpu.VMEM((CHUNK, dd), jnp.float32),
            pltpu.VMEM_SHARED((N, dd), jnp.float32),
            pltpu.SemaphoreType.DMA,
        ],
    )(functools.partial(_edge_body, dd))


_edge128 = _make_edge_kernel(D)


def _norms_body(degp_ref, ns_ref, nd_ref):
    dout = degp_ref[0, 0] + degp_ref[1, 0]
    din = degp_ref[0, 1] + degp_ref[1, 1]
    ns_ref[...] = jnp.where(dout > 0, lax.rsqrt(dout), 0.0)
    nd_ref[...] = jnp.where(din > 0, lax.rsqrt(din), 0.0)


def _norms(degp):
    return pl.pallas_call(
        _norms_body,
        out_shape=(jax.ShapeDtypeStruct((N, 1), jnp.float32),
                   jax.ShapeDtypeStruct((N, 1), jnp.float32)),
    )(degp.reshape(NC, 2, N, 1))


def _first_body(feats_ref, ns_ref, w_ref, out_ref):
    h = feats_ref[...] * ns_ref[...]
    out_ref[...] = jnp.dot(h, w_ref[...], preferred_element_type=jnp.float32)


def _first_tc(feats, ns, w):
    return pl.pallas_call(
        _first_body,
        out_shape=jax.ShapeDtypeStruct((N, D), jnp.float32),
    )(feats, ns, w)


def _mid_body(aggp_ref, nd_ref, ns_ref, w_ref, b_ref, sb_ref, out_ref):
    agg = aggp_ref[0] + aggp_ref[1]
    h = agg * nd_ref[...] + b_ref[...] + sb_ref[0, 0]
    h = jnp.maximum(h, 0.0) * ns_ref[...]
    out_ref[...] = jnp.dot(h, w_ref[...], preferred_element_type=jnp.float32)


def _mid_tc(aggp, nd, ns, w, b, sb):
    dout = w.shape[1]
    return pl.pallas_call(
        _mid_body,
        out_shape=jax.ShapeDtypeStruct((N, dout), jnp.float32),
    )(aggp, nd, ns, w, b.reshape(1, -1), sb.reshape(1, 1))


def _final_body(aggp_ref, nd_ref, b_ref, sb_ref, out_ref):
    agg = aggp_ref[0] + aggp_ref[1]
    h = agg * nd_ref[...] + b_ref[...] + sb_ref[0, 0]
    h = jnp.maximum(h, 0.0)
    col = lax.broadcasted_iota(jnp.int32, h.shape, 1)
    h = jnp.where(col < C, h, -1e30)
    m = jnp.max(h, axis=1, keepdims=True)
    e = jnp.exp(h - m)
    p = e / jnp.sum(e, axis=1, keepdims=True)
    out_ref[...] = p[:, :C]


def _final_tc(aggp, nd, b, sb):
    return pl.pallas_call(
        _final_body,
        out_shape=jax.ShapeDtypeStruct((N, C), jnp.float32),
    )(aggp, nd, b.reshape(1, -1), sb.reshape(1, 1))


def kernel(feats, edge_index, Ws, W_last, bs, b_last, sbs):
    ei = edge_index.astype(jnp.int32)
    src3 = ei[0].reshape(NW, NCHUNK, CHUNK)
    dst3 = ei[1].reshape(NW, NCHUNK, CHUNK)
    zeros_nd = jnp.zeros((N, D), jnp.float32)
    w_last_p = jnp.pad(W_last, ((0, 0), (0, CP - C)))
    b_last_p = jnp.pad(b_last, (0, CP - C))

    degp = _deg_kernel(src3, dst3, zeros_nd[:, 0])
    ns, nd = _norms(degp)

    table = _first_tc(feats, ns, Ws[0])
    aggp = _edge128(table, src3, dst3, zeros_nd)
    for i in range(1, 9):
        table = _mid_tc(aggp, nd, ns, Ws[i], bs[i - 1], sbs[i - 1])
        aggp = _edge128(table, src3, dst3, zeros_nd)
    table = _mid_tc(aggp, nd, ns, w_last_p, bs[8], sbs[8])
    aggp = _edge128(table, src3, dst3, zeros_nd)
    return _final_tc(aggp, nd, b_last_p, sbs[9])


# speedup vs baseline: 8.3550x; 1.3008x over previous
import functools

import jax
import jax.numpy as jnp
from jax import lax
from jax.experimental import pallas as pl
from jax.experimental.pallas import tpu as pltpu
from jax.experimental.pallas import tpu_sc as plsc

N = 10000
E = 320000
D = 128
C = 40
CP = 128

NC = 2
NS = 16
NW = NC * NS
EPW = E // NW
CHUNK = 80
NCHUNK = EPW // CHUNK
NBUF = 2
NG = NCHUNK // NBUF
DCHUNK = 80
DNCHUNK = EPW // DCHUNK
NPS_A = 624
NPS_LAST = N - 15 * NPS_A

_MESH = plsc.VectorSubcoreMesh(core_axis_name="c", subcore_axis_name="s")


def _deg_body(src_hbm, dst_hbm, zeros_hbm, out_hbm,
              src_v, dst_v, ones_v, dout_sh, din_sh):
    cid = lax.axis_index("c")
    sid = lax.axis_index("s")
    wid = sid * NC + cid

    @pl.when(sid == 0)
    def _():
        pltpu.sync_copy(zeros_hbm.at[pl.ds(0, N)], dout_sh)

    @pl.when(sid == 1)
    def _():
        pltpu.sync_copy(zeros_hbm.at[pl.ds(0, N)], din_sh)

    pltpu.sync_copy(src_hbm.at[wid], src_v)
    pltpu.sync_copy(dst_hbm.at[wid], dst_v)
    for k in range(DCHUNK // 16):
        ones_v[pl.ds(k * 16, 16)] = jnp.full((16,), 1.0, jnp.float32)
    plsc.subcore_barrier()

    def body(j, carry):
        pltpu.sync_copy(ones_v, dout_sh.at[src_v.at[j]], add=True)
        pltpu.sync_copy(ones_v, din_sh.at[dst_v.at[j]], add=True)
        return carry

    lax.fori_loop(0, DNCHUNK, body, 0)
    plsc.subcore_barrier()

    @pl.when(sid == 0)
    def _():
        pltpu.sync_copy(dout_sh, out_hbm.at[cid, 0])

    @pl.when(sid == 1)
    def _():
        pltpu.sync_copy(din_sh, out_hbm.at[cid, 1])


_deg_kernel = functools.partial(
    pl.kernel,
    out_type=jax.ShapeDtypeStruct((NC, 2, N), jnp.float32),
    mesh=_MESH,
    scratch_types=[
        pltpu.VMEM((DNCHUNK, DCHUNK), jnp.int32),
        pltpu.VMEM((DNCHUNK, DCHUNK), jnp.int32),
        pltpu.VMEM((DCHUNK,), jnp.float32),
        pltpu.VMEM_SHARED((N,), jnp.float32),
        pltpu.VMEM_SHARED((N,), jnp.float32),
    ],
)(_deg_body)


def _edge_body(dd, table_hbm, esrc_hbm, edst_hbm, zeros_hbm, out_hbm,
               src_v, dst_v, buf_v, agg_sh, *sems):
    gs, ss = sems[:NBUF], sems[NBUF:]
    cid = lax.axis_index("c")
    sid = lax.axis_index("s")
    wid = sid * NC + cid

    base = pl.multiple_of(sid * NPS_A, 8)

    @pl.when(sid < 15)
    def _():
        pltpu.sync_copy(zeros_hbm.at[pl.ds(base, NPS_A)],
                        agg_sh.at[pl.ds(base, NPS_A)])

    @pl.when(sid == 15)
    def _():
        pltpu.sync_copy(zeros_hbm.at[pl.ds(15 * NPS_A, NPS_LAST)],
                        agg_sh.at[pl.ds(15 * NPS_A, NPS_LAST)])

    pltpu.sync_copy(esrc_hbm.at[wid], src_v)
    pltpu.sync_copy(edst_hbm.at[wid], dst_v)
    plsc.subcore_barrier()

    def fire_gather(j, b):
        off = pl.multiple_of(j * CHUNK, 8)
        pltpu.async_copy(table_hbm.at[src_v.at[pl.ds(off, CHUNK)]],
                         buf_v.at[b], gs[b])

    def wait_chunk(sem):
        pltpu.make_async_copy(table_hbm.at[src_v.at[pl.ds(0, CHUNK)]],
                              buf_v.at[0], sem).wait()

    for b in range(NBUF):
        fire_gather(b, b)

    def grp(g, carry):
        j0 = g * NBUF
        for b in range(NBUF):
            wait_chunk(gs[b])
            pltpu.async_copy(buf_v.at[b], agg_sh.at[dst_v.at[j0 + b]],
                             ss[b], add=True)

        @pl.when(g < NG - 1)
        def _():
            for b in range(NBUF):
                wait_chunk(ss[b])
                fire_gather(j0 + NBUF + b, b)

        return carry

    lax.fori_loop(0, NG, grp, 0)
    for b in range(NBUF):
        wait_chunk(ss[b])
    for j in range(NG * NBUF, NCHUNK):
        fire_gather(j, 0)
        wait_chunk(gs[0])
        pltpu.async_copy(buf_v.at[0], agg_sh.at[dst_v.at[j]], ss[0], add=True)
        wait_chunk(ss[0])
    plsc.subcore_barrier()

    @pl.when(sid < 15)
    def _():
        pltpu.sync_copy(agg_sh.at[pl.ds(base, NPS_A)],
                        out_hbm.at[cid, pl.ds(base, NPS_A)])

    @pl.when(sid == 15)
    def _():
        pltpu.sync_copy(agg_sh.at[pl.ds(15 * NPS_A, NPS_LAST)],
                        out_hbm.at[cid, pl.ds(15 * NPS_A, NPS_LAST)])


def _make_edge_kernel(dd):
    return functools.partial(
        pl.kernel,
        out_type=jax.ShapeDtypeStruct((NC, N, dd), jnp.float32),
        mesh=_MESH,
        scratch_types=[
            pltpu.VMEM((EPW,), jnp.int32),
            pltpu.VMEM((NCHUNK, CHUNK), jnp.int32),
            pltpu.VMEM((NBUF, CHUNK, dd), jnp.float32),
            pltpu.VMEM_SHARED((N, dd), jnp.float32),
        ] + [pltpu.SemaphoreType.DMA] * (2 * NBUF),
    )(functools.partial(_edge_body, dd))


_edge128 = _make_edge_kernel(D)


def _norms_body(degp_ref, ns_ref, nd_ref):
    dout = degp_ref[0, 0] + degp_ref[1, 0]
    din = degp_ref[0, 1] + degp_ref[1, 1]
    ns_ref[...] = jnp.where(dout > 0, lax.rsqrt(dout), 0.0)
    nd_ref[...] = jnp.where(din > 0, lax.rsqrt(din), 0.0)


def _norms(degp):
    return pl.pallas_call(
        _norms_body,
        out_shape=(jax.ShapeDtypeStruct((N, 1), jnp.float32),
                   jax.ShapeDtypeStruct((N, 1), jnp.float32)),
    )(degp.reshape(NC, 2, N, 1))


def _first_body(feats_ref, ns_ref, w_ref, out_ref):
    h = feats_ref[...] * ns_ref[...]
    out_ref[...] = jnp.dot(h, w_ref[...], preferred_element_type=jnp.float32)


def _first_tc(feats, ns, w):
    return pl.pallas_call(
        _first_body,
        out_shape=jax.ShapeDtypeStruct((N, D), jnp.float32),
    )(feats, ns, w)


def _mid_body(aggp_ref, nd_ref, ns_ref, w_ref, b_ref, sb_ref, out_ref):
    agg = aggp_ref[0] + aggp_ref[1]
    h = agg * nd_ref[...] + b_ref[...] + sb_ref[0, 0]
    h = jnp.maximum(h, 0.0) * ns_ref[...]
    out_ref[...] = jnp.dot(h, w_ref[...], preferred_element_type=jnp.float32)


def _mid_tc(aggp, nd, ns, w, b, sb):
    dout = w.shape[1]
    return pl.pallas_call(
        _mid_body,
        out_shape=jax.ShapeDtypeStruct((N, dout), jnp.float32),
    )(aggp, nd, ns, w, b.reshape(1, -1), sb.reshape(1, 1))


def _final_body(aggp_ref, nd_ref, b_ref, sb_ref, out_ref):
    agg = aggp_ref[0] + aggp_ref[1]
    h = agg * nd_ref[...] + b_ref[...] + sb_ref[0, 0]
    h = jnp.maximum(h, 0.0)
    col = lax.broadcasted_iota(jnp.int32, h.shape, 1)
    h = jnp.where(col < C, h, -1e30)
    m = jnp.max(h, axis=1, keepdims=True)
    e = jnp.exp(h - m)
    p = e / jnp.sum(e, axis=1, keepdims=True)
    out_ref[...] = p[:, :C]


def _final_tc(aggp, nd, b, sb):
    return pl.pallas_call(
        _final_body,
        out_shape=jax.ShapeDtypeStruct((N, C), jnp.float32),
    )(aggp, nd, b.reshape(1, -1), sb.reshape(1, 1))


def kernel(feats, edge_index, Ws, W_last, bs, b_last, sbs):
    ei = edge_index.astype(jnp.int32)
    esrc = ei[0].reshape(NW, EPW)
    edst = ei[1].reshape(NW, NCHUNK, CHUNK)
    srcd = ei[0].reshape(NW, DNCHUNK, DCHUNK)
    dstd = ei[1].reshape(NW, DNCHUNK, DCHUNK)
    zeros_nd = jnp.zeros((N, D), jnp.float32)
    w_last_p = jnp.pad(W_last, ((0, 0), (0, CP - C)))
    b_last_p = jnp.pad(b_last, (0, CP - C))

    degp = _deg_kernel(srcd, dstd, zeros_nd[:, 0])
    ns, nd = _norms(degp)

    table = _first_tc(feats, ns, Ws[0])
    aggp = _edge128(table, esrc, edst, zeros_nd)
    for i in range(1, 9):
        table = _mid_tc(aggp, nd, ns, Ws[i], bs[i - 1], sbs[i - 1])
        aggp = _edge128(table, esrc, edst, zeros_nd)
    table = _mid_tc(aggp, nd, ns, w_last_p, bs[8], sbs[8])
    aggp = _edge128(table, esrc, edst, zeros_nd)
    return _final_tc(aggp, nd, b_last_p, sbs[9])
